# baseline (speedup 1.0000x reference)
import jax
import jax.numpy as jnp
from jax import lax
from jax.experimental import pallas as pl
from jax.experimental.pallas import tpu as pltpu
from jax.experimental.pallas import tpu_sc as plsc

N = 10000
E = 320000
F = 128
K = 5
NP = 10240
NPAD = N

NC = 2
NS = 16
NW = NC * NS
CHUNK = 128
SLICE = NP // NS

NCH = 80
TOTCH = NW * NCH
EP = TOTCH * CHUNK
_Q = 40


def _sc_mesh():
    return plsc.VectorSubcoreMesh(
        core_axis_name="c", subcore_axis_name="s",
        num_cores=NC, num_subcores=NS)


def _sc_deg(rowm, colm):
    def body(rowm_hbm, colm_hbm, deg2_hbm, rowp_hbm,
             row_v, col_v, rowp_v, deg_l, sumbuf, res_v, deg_sh):
        zeros16 = jnp.zeros((16,), jnp.float32)
        ones16 = jnp.ones((16,), jnp.float32)
        cid = lax.axis_index("c")
        sid = lax.axis_index("s")
        wid = cid * NS + sid
        base = wid * NCH

        pltpu.sync_copy(rowm_hbm.at[pl.ds(base, NCH)], row_v)
        pltpu.sync_copy(colm_hbm.at[pl.ds(base, NCH)], col_v)

        def zbody(i, _):
            deg_l[pl.ds(i * 16, 16)] = zeros16
            return 0
        lax.fori_loop(0, NP // 16, zbody, 0)

        def chunk_body(j, _):
            for t in range(CHUNK // 16):
                r16 = row_v[j, pl.ds(t * 16, 16)]
                c16 = col_v[j, pl.ds(t * 16, 16)]
                m = r16 != c16
                rowp_v[j, pl.ds(t * 16, 16)] = jnp.where(m, r16, NPAD)
                plsc.addupdate_scatter(deg_l, [r16], ones16, mask=m)
            return 0
        lax.fori_loop(0, NCH, chunk_body, 0)

        pltpu.sync_copy(rowp_v, rowp_hbm.at[pl.ds(base, NCH)])

        pltpu.sync_copy(deg_l, deg_sh.at[sid])
        plsc.subcore_barrier()
        pltpu.sync_copy(deg_sh.at[:, pl.ds(sid * SLICE, SLICE)], sumbuf)

        def sbody(g, _):
            acc = sumbuf[0, pl.ds(g * 16, 16)]
            for r in range(1, NS):
                acc = acc + sumbuf[r, pl.ds(g * 16, 16)]
            res_v[pl.ds(g * 16, 16)] = acc
            return 0
        lax.fori_loop(0, SLICE // 16, sbody, 0)

        pltpu.sync_copy(res_v,
                        deg2_hbm.at[pl.ds(cid * NP + sid * SLICE, SLICE)])

    return pl.kernel(
        body,
        out_type=(jax.ShapeDtypeStruct((NC * NP,), jnp.float32),
                  jax.ShapeDtypeStruct((TOTCH, CHUNK), jnp.int32)),
        mesh=_sc_mesh(),
        compiler_params=pltpu.CompilerParams(needs_layout_passes=False),
        scratch_types=[
            pltpu.VMEM((NCH, CHUNK), jnp.int32),
            pltpu.VMEM((NCH, CHUNK), jnp.int32),
            pltpu.VMEM((NCH, CHUNK), jnp.int32),
            pltpu.VMEM((NP,), jnp.float32),
            pltpu.VMEM((NS, SLICE), jnp.float32),
            pltpu.VMEM((SLICE,), jnp.float32),
            pltpu.VMEM_SHARED((NS, NP), jnp.float32),
        ],
    )(rowm, colm)


def _sc_prop(g, rowp, colm):
    def body(g_hbm, rowp_hbm, colm_hbm, acc2_hbm,
             rowi_v, coli_v, rows_a, rows_b, gsem_a, gsem_b, acc_sh):
        zeros16 = jnp.zeros((16,), jnp.float32)
        cid = lax.axis_index("c")
        sid = lax.axis_index("s")
        wid = cid * NS + sid
        base = wid * NCH

        def zbody(r, _):
            for t in range(F // 16):
                rows_a[r, pl.ds(t * 16, 16)] = zeros16
            return 0
        lax.fori_loop(0, CHUNK, zbody, 0)
        for kk in range(SLICE // CHUNK):
            pltpu.sync_copy(
                rows_a, acc_sh.at[pl.ds(sid * SLICE + kk * CHUNK, CHUNK)])
        plsc.subcore_barrier()

        def gfire(j, buf, sem):
            pltpu.async_copy(g_hbm.at[rowi_v.at[j]], buf, sem)

        def gwait(j, buf, sem):
            pltpu.make_async_copy(g_hbm.at[rowi_v.at[j]], buf, sem).wait()

        def scat(j, buf):
            pltpu.sync_copy(buf, acc_sh.at[coli_v.at[j]], add=True)

        for q in range(NCH // _Q):
            cb = base + q * _Q
            pltpu.sync_copy(rowp_hbm.at[pl.ds(cb, _Q)], rowi_v)
            pltpu.sync_copy(colm_hbm.at[pl.ds(cb, _Q)], coli_v)
            gfire(0, rows_a, gsem_a)

            def pair_body(p, _):
                j0 = 2 * p
                j1 = j0 + 1
                gfire(j1, rows_b, gsem_b)
                gwait(j0, rows_a, gsem_a)
                scat(j0, rows_a)

                @pl.when(j1 + 1 < _Q)
                def _():
                    gfire(j1 + 1, rows_a, gsem_a)
                gwait(j1, rows_b, gsem_b)
                scat(j1, rows_b)
                return 0
            lax.fori_loop(0, _Q // 2, pair_body, 0)

        plsc.subcore_barrier()
        pltpu.sync_copy(acc_sh.at[pl.ds(sid * SLICE, SLICE)],
                        acc2_hbm.at[cid, pl.ds(sid * SLICE, SLICE)])

    return pl.kernel(
        body,
        out_type=jax.ShapeDtypeStruct((NC, NP, F), jnp.float32),
        mesh=_sc_mesh(),
        compiler_params=pltpu.CompilerParams(needs_layout_passes=False),
        scratch_types=[
            pltpu.VMEM((_Q, CHUNK), jnp.int32),
            pltpu.VMEM((_Q, CHUNK), jnp.int32),
            pltpu.VMEM((CHUNK, F), jnp.float32),
            pltpu.VMEM((CHUNK, F), jnp.float32),
            pltpu.SemaphoreType.DMA,
            pltpu.SemaphoreType.DMA,
            pltpu.VMEM_SHARED((NP, F), jnp.float32),
        ],
    )(g, rowp, colm)


_BLK = 512


def _tc_prep(deg2, xp):
    def body(deg_ref, x_ref, dis_ref, g_ref):
        deg = deg_ref[0, :] + deg_ref[1, :]
        dis = jnp.where(deg > 0, lax.rsqrt(deg), 0.0)
        disb = jnp.broadcast_to(dis[:, None], (_BLK, F))
        dis_ref[...] = disb
        g_ref[...] = disb * x_ref[...]

    return pl.pallas_call(
        body,
        grid=(NP // _BLK,),
        in_specs=[
            pl.BlockSpec((NC, _BLK), lambda i: (0, i)),
            pl.BlockSpec((_BLK, F), lambda i: (i, 0)),
        ],
        out_specs=[
            pl.BlockSpec((_BLK, F), lambda i: (i, 0)),
            pl.BlockSpec((_BLK, F), lambda i: (i, 0)),
        ],
        out_shape=(jax.ShapeDtypeStruct((NP, F), jnp.float32),
                   jax.ShapeDtypeStruct((NP, F), jnp.float32)),
    )(deg2, xp)


def _tc_comb1(acc2, disb):
    def body(a_ref, d_ref, tx_ref, g_ref):
        p = a_ref[0] + a_ref[1]
        d = d_ref[...]
        tx = -d * p
        tx_ref[...] = tx
        g_ref[...] = d * tx

    return pl.pallas_call(
        body,
        grid=(NP // _BLK,),
        in_specs=[
            pl.BlockSpec((NC, _BLK, F), lambda i: (0, i, 0)),
            pl.BlockSpec((_BLK, F), lambda i: (i, 0)),
        ],
        out_specs=[
            pl.BlockSpec((_BLK, F), lambda i: (i, 0)),
            pl.BlockSpec((_BLK, F), lambda i: (i, 0)),
        ],
        out_shape=(jax.ShapeDtypeStruct((NP, F), jnp.float32),
                   jax.ShapeDtypeStruct((NP, F), jnp.float32)),
    )(acc2, disb)


def _tc_comb(acc2, disb, txm2):
    def body(a_ref, d_ref, t_ref, tx_ref, g_ref):
        p = a_ref[0] + a_ref[1]
        d = d_ref[...]
        tx = -2.0 * d * p - t_ref[...]
        tx_ref[...] = tx
        g_ref[...] = d * tx

    return pl.pallas_call(
        body,
        grid=(NP // _BLK,),
        in_specs=[
            pl.BlockSpec((NC, _BLK, F), lambda i: (0, i, 0)),
            pl.BlockSpec((_BLK, F), lambda i: (i, 0)),
            pl.BlockSpec((_BLK, F), lambda i: (i, 0)),
        ],
        out_specs=[
            pl.BlockSpec((_BLK, F), lambda i: (i, 0)),
            pl.BlockSpec((_BLK, F), lambda i: (i, 0)),
        ],
        out_shape=(jax.ShapeDtypeStruct((NP, F), jnp.float32),
                   jax.ShapeDtypeStruct((NP, F), jnp.float32)),
    )(acc2, disb, txm2)


_MBLK = 1024


def _tc_final(txs, W, b2):
    def body(t0, t1, t2, t3, t4, w_ref, b_ref, o_ref):
        acc = b_ref[...].astype(jnp.float32)
        for k, t in enumerate((t0, t1, t2, t3, t4)):
            acc = acc + jnp.dot(t[...], w_ref[k],
                                preferred_element_type=jnp.float32)
        o_ref[...] = jnp.maximum(acc, 0.0)

    return pl.pallas_call(
        body,
        grid=(NP // _MBLK,),
        in_specs=[pl.BlockSpec((_MBLK, F), lambda i: (i, 0))] * K
        + [pl.BlockSpec((K, F, F), lambda i: (0, 0, 0)),
           pl.BlockSpec((1, F), lambda i: (0, 0))],
        out_specs=pl.BlockSpec((_MBLK, F), lambda i: (i, 0)),
        out_shape=jax.ShapeDtypeStruct((NP, F), jnp.float32),
    )(*txs, W, b2)


def kernel(x, edge_index, W, b):
    pad = jnp.full((EP - E,), NPAD, jnp.int32)
    rowm = jnp.concatenate([edge_index[0], pad]).reshape(TOTCH, CHUNK)
    colm = jnp.concatenate([edge_index[1], pad]).reshape(TOTCH, CHUNK)
    xp = jnp.pad(x, ((0, NP - N), (0, 0)))
    b2 = b.reshape(1, F)

    deg2, rowp = _sc_deg(rowm, colm)
    disb, g = _tc_prep(deg2.reshape(NC, NP), xp)

    acc2 = _sc_prop(g, rowp, colm)
    tx1, g = _tc_comb1(acc2, disb)

    txs = [xp, tx1]
    for _ in range(2, K):
        acc2 = _sc_prop(g, rowp, colm)
        txk, g = _tc_comb(acc2, disb, txs[-2])
        txs.append(txk)

    out = _tc_final(txs, W, b2)
    return out[:N]


# speedup vs baseline: 1.5836x; 1.5836x over previous
import jax
import jax.numpy as jnp
from jax import lax
from jax.experimental import pallas as pl
from jax.experimental.pallas import tpu as pltpu
from jax.experimental.pallas import tpu_sc as plsc

N = 10000
E = 320000
F = 128
K = 5
NP = 10240
NPAD = N

NC = 2
NS = 16
NW = NC * NS
CHUNK = 128
SLICE = NP // NS

MAXCH = (E + NW * CHUNK - 1) // (NW * CHUNK)
EP = NW * MAXCH * CHUNK
_HALF = 40


def _sc_mesh():
    return plsc.VectorSubcoreMesh(
        core_axis_name="c", subcore_axis_name="s",
        num_cores=NC, num_subcores=NS)


def _sc_deg(rowm, colm):
    def body(rowm_hbm, colm_hbm, deg2_hbm, rowp_hbm,
             row_v, col_v, rowp_v, deg_l, sumbuf, res_v, deg_sh):
        zeros16 = jnp.zeros((16,), jnp.float32)
        ones16 = jnp.ones((16,), jnp.float32)
        cid = lax.axis_index("c")
        sid = lax.axis_index("s")
        wid = cid * NS + sid

        pltpu.sync_copy(rowm_hbm.at[wid], row_v)
        pltpu.sync_copy(colm_hbm.at[wid], col_v)

        def zbody(i, _):
            deg_l[pl.ds(i * 16, 16)] = zeros16
            return 0
        lax.fori_loop(0, NP // 16, zbody, 0)

        def chunk_body(j, _):
            for t in range(CHUNK // 16):
                r16 = row_v[j, pl.ds(t * 16, 16)]
                c16 = col_v[j, pl.ds(t * 16, 16)]
                m = r16 != c16
                rowp_v[j, pl.ds(t * 16, 16)] = jnp.where(m, r16, NPAD)
                plsc.addupdate_scatter(deg_l, [r16], ones16, mask=m)
            return 0
        lax.fori_loop(0, MAXCH, chunk_body, 0)

        pltpu.sync_copy(rowp_v, rowp_hbm.at[wid])

        pltpu.sync_copy(deg_l, deg_sh.at[sid])
        plsc.subcore_barrier()
        pltpu.sync_copy(deg_sh.at[:, pl.ds(sid * SLICE, SLICE)], sumbuf)

        def sbody(g, _):
            acc = sumbuf[0, pl.ds(g * 16, 16)]
            for r in range(1, NS):
                acc = acc + sumbuf[r, pl.ds(g * 16, 16)]
            res_v[pl.ds(g * 16, 16)] = acc
            return 0
        lax.fori_loop(0, SLICE // 16, sbody, 0)

        pltpu.sync_copy(res_v,
                        deg2_hbm.at[pl.ds(cid * NP + sid * SLICE, SLICE)])

    return pl.kernel(
        body,
        out_type=(jax.ShapeDtypeStruct((NC * NP,), jnp.float32),
                  jax.ShapeDtypeStruct((NW, MAXCH, CHUNK), jnp.int32)),
        mesh=_sc_mesh(),
        compiler_params=pltpu.CompilerParams(needs_layout_passes=False),
        scratch_types=[
            pltpu.VMEM((MAXCH, CHUNK), jnp.int32),
            pltpu.VMEM((MAXCH, CHUNK), jnp.int32),
            pltpu.VMEM((MAXCH, CHUNK), jnp.int32),
            pltpu.VMEM((NP,), jnp.float32),
            pltpu.VMEM((NS, SLICE), jnp.float32),
            pltpu.VMEM((SLICE,), jnp.float32),
            pltpu.VMEM_SHARED((NS, NP), jnp.float32),
        ],
    )(rowm, colm)


def _sc_prop(g, rowp, colm):
    def body(g_hbm, rowp_hbm, colm_hbm, acc2_hbm,
             rowi_v, coli_v, rows_a, rows_b, gsem_a, gsem_b, acc_sh):
        zeros16 = jnp.zeros((16,), jnp.float32)
        cid = lax.axis_index("c")
        sid = lax.axis_index("s")
        wid = cid * NS + sid

        def zbody(r, _):
            for t in range(F // 16):
                rows_a[r, pl.ds(t * 16, 16)] = zeros16
            return 0
        lax.fori_loop(0, CHUNK, zbody, 0)
        for kk in range(SLICE // CHUNK):
            pltpu.sync_copy(
                rows_a, acc_sh.at[pl.ds(sid * SLICE + kk * CHUNK, CHUNK)])
        plsc.subcore_barrier()

        def gfire(j, buf, sem):
            pltpu.async_copy(g_hbm.at[rowi_v.at[j]], buf, sem)

        def gwait(j, buf, sem):
            pltpu.make_async_copy(g_hbm.at[rowi_v.at[j]], buf, sem).wait()

        def scat(j, buf):
            pltpu.sync_copy(buf, acc_sh.at[coli_v.at[j]], add=True)

        for cbase, hcnt in ((0, _HALF), (_HALF, MAXCH - _HALF)):
            pltpu.sync_copy(rowp_hbm.at[wid, pl.ds(cbase, hcnt)],
                            rowi_v.at[pl.ds(0, hcnt)])
            pltpu.sync_copy(colm_hbm.at[wid, pl.ds(cbase, hcnt)],
                            coli_v.at[pl.ds(0, hcnt)])
            gfire(0, rows_a, gsem_a)

            def pair_body(p, _):
                j0 = 2 * p
                j1 = j0 + 1

                @pl.when(j1 < hcnt)
                def _():
                    gfire(j1, rows_b, gsem_b)
                gwait(j0, rows_a, gsem_a)
                scat(j0, rows_a)

                @pl.when(j1 + 1 < hcnt)
                def _():
                    gfire(j1 + 1, rows_a, gsem_a)

                @pl.when(j1 < hcnt)
                def _():
                    gwait(j1, rows_b, gsem_b)
                    scat(j1, rows_b)
                return 0
            lax.fori_loop(0, (hcnt + 1) // 2, pair_body, 0)

        plsc.subcore_barrier()
        pltpu.sync_copy(acc_sh.at[pl.ds(sid * SLICE, SLICE)],
                        acc2_hbm.at[cid, pl.ds(sid * SLICE, SLICE)])

    return pl.kernel(
        body,
        out_type=jax.ShapeDtypeStruct((NC, NP, F), jnp.float32),
        mesh=_sc_mesh(),
        compiler_params=pltpu.CompilerParams(needs_layout_passes=False),
        scratch_types=[
            pltpu.VMEM((_HALF, CHUNK), jnp.int32),
            pltpu.VMEM((_HALF, CHUNK), jnp.int32),
            pltpu.VMEM((CHUNK, F), jnp.float32),
            pltpu.VMEM((CHUNK, F), jnp.float32),
            pltpu.SemaphoreType.DMA,
            pltpu.SemaphoreType.DMA,
            pltpu.VMEM_SHARED((NP, F), jnp.float32),
        ],
    )(g, rowp, colm)


_BLK = 512


def _tc_prep(deg2, xp):
    def body(deg_ref, x_ref, dis_ref, g_ref):
        deg = deg_ref[0, :] + deg_ref[1, :]
        dis = jnp.where(deg > 0, lax.rsqrt(deg), 0.0)
        disb = jnp.broadcast_to(dis[:, None], (_BLK, F))
        dis_ref[...] = disb
        g_ref[...] = disb * x_ref[...]

    return pl.pallas_call(
        body,
        grid=(NP // _BLK,),
        in_specs=[
            pl.BlockSpec((NC, _BLK), lambda i: (0, i)),
            pl.BlockSpec((_BLK, F), lambda i: (i, 0)),
        ],
        out_specs=[
            pl.BlockSpec((_BLK, F), lambda i: (i, 0)),
            pl.BlockSpec((_BLK, F), lambda i: (i, 0)),
        ],
        out_shape=(jax.ShapeDtypeStruct((NP, F), jnp.float32),
                   jax.ShapeDtypeStruct((NP, F), jnp.float32)),
    )(deg2, xp)


def _tc_comb1(acc2, disb):
    def body(a_ref, d_ref, tx_ref, g_ref):
        p = a_ref[0] + a_ref[1]
        d = d_ref[...]
        tx = -d * p
        tx_ref[...] = tx
        g_ref[...] = d * tx

    return pl.pallas_call(
        body,
        grid=(NP // _BLK,),
        in_specs=[
            pl.BlockSpec((NC, _BLK, F), lambda i: (0, i, 0)),
            pl.BlockSpec((_BLK, F), lambda i: (i, 0)),
        ],
        out_specs=[
            pl.BlockSpec((_BLK, F), lambda i: (i, 0)),
            pl.BlockSpec((_BLK, F), lambda i: (i, 0)),
        ],
        out_shape=(jax.ShapeDtypeStruct((NP, F), jnp.float32),
                   jax.ShapeDtypeStruct((NP, F), jnp.float32)),
    )(acc2, disb)


def _tc_comb(acc2, disb, txm2):
    def body(a_ref, d_ref, t_ref, tx_ref, g_ref):
        p = a_ref[0] + a_ref[1]
        d = d_ref[...]
        tx = -2.0 * d * p - t_ref[...]
        tx_ref[...] = tx
        g_ref[...] = d * tx

    return pl.pallas_call(
        body,
        grid=(NP // _BLK,),
        in_specs=[
            pl.BlockSpec((NC, _BLK, F), lambda i: (0, i, 0)),
            pl.BlockSpec((_BLK, F), lambda i: (i, 0)),
            pl.BlockSpec((_BLK, F), lambda i: (i, 0)),
        ],
        out_specs=[
            pl.BlockSpec((_BLK, F), lambda i: (i, 0)),
            pl.BlockSpec((_BLK, F), lambda i: (i, 0)),
        ],
        out_shape=(jax.ShapeDtypeStruct((NP, F), jnp.float32),
                   jax.ShapeDtypeStruct((NP, F), jnp.float32)),
    )(acc2, disb, txm2)


_MBLK = 1024


def _tc_final(txs, W, b2):
    def body(t0, t1, t2, t3, t4, w_ref, b_ref, o_ref):
        acc = b_ref[...].astype(jnp.float32)
        for k, t in enumerate((t0, t1, t2, t3, t4)):
            acc = acc + jnp.dot(t[...], w_ref[k],
                                preferred_element_type=jnp.float32)
        o_ref[...] = jnp.maximum(acc, 0.0)

    return pl.pallas_call(
        body,
        grid=(NP // _MBLK,),
        in_specs=[pl.BlockSpec((_MBLK, F), lambda i: (i, 0))] * K
        + [pl.BlockSpec((K, F, F), lambda i: (0, 0, 0)),
           pl.BlockSpec((1, F), lambda i: (0, 0))],
        out_specs=pl.BlockSpec((_MBLK, F), lambda i: (i, 0)),
        out_shape=jax.ShapeDtypeStruct((NP, F), jnp.float32),
    )(*txs, W, b2)


def kernel(x, edge_index, W, b):
    pad = jnp.full((EP - E,), NPAD, jnp.int32)
    rowm = jnp.concatenate([edge_index[0], pad]).reshape(NW, MAXCH, CHUNK)
    colm = jnp.concatenate([edge_index[1], pad]).reshape(NW, MAXCH, CHUNK)
    xp = jnp.pad(x, ((0, NP - N), (0, 0)))
    b2 = b.reshape(1, F)

    deg2, rowp = _sc_deg(rowm, colm)
    disb, g = _tc_prep(deg2.reshape(NC, NP), xp)

    acc2 = _sc_prop(g, rowp, colm)
    tx1, g = _tc_comb1(acc2, disb)

    txs = [xp, tx1]
    for _ in range(2, K):
        acc2 = _sc_prop(g, rowp, colm)
        txk, g = _tc_comb(acc2, disb, txs[-2])
        txs.append(txk)

    out = _tc_final(txs, W, b2)
    return out[:N]
